# Initial kernel scaffold; baseline (speedup 1.0000x reference)
import jax
import jax.numpy as jnp
from jax.experimental import pallas as pl


def kernel(x, edge_index, conv_h_W, conv_h_b, conv_x_W, conv_x_b, gate_f_W, gate_f_b, gate_u_W, gate_u_b, ff1_W, ff1_b, ff2_W, ff2_b, clf1_W, clf1_b, clf2_W, clf2_b):
    raise NotImplementedError("write your pallas kernel here")


# speedup vs baseline: 32.6018x; 32.6018x over previous
import functools

import jax
import jax.numpy as jnp
from jax import lax
from jax.experimental import pallas as pl
from jax.experimental.pallas import tpu as pltpu
from jax.experimental.pallas import tpu_sc as plsc

N = 10000
E = 320000
D = 128
NPAD = 10240
NC = 2
NS = 16
NW = NC * NS
CHUNK = 128
CPT = -(-E // (NW * CHUNK))
EPAD = NW * CPT * CHUNK
RPT = NPAD // NS
ALPHA = 0.2
BR = 1024
GRID = NPAD // BR

@functools.cache
def _sc_kernels():
    mesh = plsc.VectorSubcoreMesh(core_axis_name="c", subcore_axis_name="s",
                                  num_cores=NC, num_subcores=NS)

    @functools.partial(
        pl.kernel,
        out_type=jax.ShapeDtypeStruct((NC, NPAD, D), jnp.float32),
        mesh=mesh,
        scratch_types=[
            pltpu.VMEM((CPT, CHUNK), jnp.int32),
            pltpu.VMEM((CPT, CHUNK), jnp.int32),
            pltpu.VMEM((CHUNK, D), jnp.float32),
            pltpu.VMEM_SHARED((NPAD, D), jnp.float32),
            pltpu.SemaphoreType.DMA,
        ],
    )
    def sc_pass(u_hbm, src_hbm, dst_hbm, zeros_hbm, out_hbm, srcb, dstb,
                rows, wsh, sem):
        c = lax.axis_index("c")
        s = lax.axis_index("s")
        wid = c * NS + s
        pltpu.sync_copy(src_hbm.at[wid], srcb)
        pltpu.sync_copy(dst_hbm.at[wid], dstb)
        base = s * RPT
        pltpu.sync_copy(zeros_hbm.at[pl.ds(base, RPT)],
                        wsh.at[pl.ds(base, RPT)])
        plsc.subcore_barrier()

        def body(j, carry):
            pltpu.async_copy(u_hbm.at[srcb.at[j]], rows, sem).wait()
            pltpu.sync_copy(rows, wsh.at[dstb.at[j]], add=True)
            return carry

        lax.fori_loop(0, CPT, body, 0)
        plsc.subcore_barrier()
        pltpu.sync_copy(wsh.at[pl.ds(base, RPT)],
                        out_hbm.at[c, pl.ds(base, RPT)])

    return sc_pass


def _ln(h):
    mu = jnp.mean(h, axis=-1, keepdims=True)
    var = jnp.mean((h - mu) * (h - mu), axis=-1, keepdims=True)
    return (h - mu) * lax.rsqrt(var + 1e-5)


def _dot(a, b):
    return jnp.dot(a, b, preferred_element_type=jnp.float32)


def _prep_body(d0, d1, x, u1_ref, dinv_ref):
    deg = d0[:, 0:1] + d1[:, 0:1] + 1.0
    dinv = lax.rsqrt(deg)
    dinv_ref[...] = dinv
    u1_ref[...] = x[...] * dinv


def _heads_accum(ah_or_none, ax, Wh, bh, Wx, bx, gf, gfb, gu, gub):
    gfh, gfx = gf[0:1, :], gf[1:2, :]
    guh, gux = gu[0:1, :], gu[1:2, :]
    acc = jnp.zeros_like(ax)
    for hd in range(4):
        if ah_or_none is None:
            hh = bh[hd:hd + 1, :]
            fh = jnp.sum(hh * gfh)
            uh = jnp.sum(hh * guh)
        else:
            hh = _dot(ah_or_none, Wh[hd]) + bh[hd:hd + 1, :]
            fh = jnp.sum(hh * gfh, axis=-1, keepdims=True)
            uh = jnp.sum(hh * guh, axis=-1, keepdims=True)
        xx = _dot(ax, Wx[hd]) + bx[hd:hd + 1, :]
        f = ALPHA * jnp.tanh(fh + jnp.sum(xx * gfx, axis=-1, keepdims=True)
                             + gfb[0, 0])
        u = ALPHA * jnp.tanh(uh + jnp.sum(xx * gux, axis=-1, keepdims=True)
                             + gub[0, 0])
        acc = acc + (1.0 + f) * hh + (1.0 + u) * xx
    return acc * 0.25


def _ffn(h, f1w, f1b, f2w, f2b):
    return _dot(jax.nn.relu(_dot(h, f1w) + f1b), f2w) + f2b


def _layer0_body(w0, w1, u1, dinv, Wx, bx, bh, gf, gfb, gu, gub,
                 f1w, f1b, f2w, f2b, ax_ref, h1_ref, u2_ref):
    dv = dinv[...]
    ax = dv * (w0[...] + w1[...] + u1[...])
    ax_ref[...] = ax
    mh = _heads_accum(None, ax, None, bh, Wx, bx, gf, gfb, gu, gub)
    h = _ln(mh)
    h1 = _ln(_ffn(h, f1w[...], f1b[...], f2w[...], f2b[...]) + h)
    h1_ref[...] = h1
    u2_ref[...] = dv * h1


def _layer1_full_body(w0, w1, u2, ax_in, h1_in, dinv, Wh, bh, Wx, bx,
                      gf, gfb, gu, gub, f1w, f1b, f2w, f2b,
                      c1w, c1b, c2w, c2b, out_ref):
    dv = dinv[...]
    ah = dv * (w0[...] + w1[...] + u2[...])
    ax = ax_in[...]
    h_prev = h1_in[...]
    mh = _heads_accum(ah, ax, Wh, bh, Wx, bx, gf, gfb, gu, gub)
    h = _ln(mh + h_prev)
    h = _ln(_ffn(h, f1w[...], f1b[...], f2w[...], f2b[...]) + h)
    out_ref[...] = _dot(jax.nn.relu(_dot(h, c1w[...]) + c1b[...]),
                        c2w[...]) + c2b[...]


def _row_spec(width):
    return pl.BlockSpec((BR, width), lambda i: (i, 0))


def _full_spec(shape):
    nd = len(shape)
    return pl.BlockSpec(shape, lambda i, _nd=nd: (0,) * _nd)


_F32 = jnp.float32


def _prep_call(d0, d1, x):
    return pl.pallas_call(
        _prep_body,
        grid=(GRID,),
        in_specs=[_row_spec(D), _row_spec(D), _row_spec(D)],
        out_specs=[_row_spec(D), _row_spec(1)],
        out_shape=[jax.ShapeDtypeStruct((NPAD, D), _F32),
                   jax.ShapeDtypeStruct((NPAD, 1), _F32)],
    )(d0, d1, x)


def _layer0_call(w0, w1, u1, dinv, Wx, bx, bh, gf, gfb, gu, gub,
                 f1w, f1b, f2w, f2b):
    return pl.pallas_call(
        _layer0_body,
        grid=(GRID,),
        in_specs=[_row_spec(D), _row_spec(D), _row_spec(D), _row_spec(1),
                  _full_spec((4, D, D)), _full_spec((4, D)), _full_spec((4, D)),
                  _full_spec((2, D)), _full_spec((1, 1)),
                  _full_spec((2, D)), _full_spec((1, 1)),
                  _full_spec((D, D)), _full_spec((1, D)),
                  _full_spec((D, D)), _full_spec((1, D))],
        out_specs=[_row_spec(D), _row_spec(D), _row_spec(D)],
        out_shape=[jax.ShapeDtypeStruct((NPAD, D), _F32)] * 3,
    )(w0, w1, u1, dinv, Wx, bx, bh, gf, gfb, gu, gub, f1w, f1b, f2w, f2b)


def _layer1_call(w0, w1, u2, ax, h1, dinv, Wh, bh, Wx, bx, gf, gfb, gu, gub,
                 f1w, f1b, f2w, f2b, c1w, c1b, c2w, c2b):
    return pl.pallas_call(
        _layer1_full_body,
        grid=(GRID,),
        in_specs=[_row_spec(D), _row_spec(D), _row_spec(D), _row_spec(D),
                  _row_spec(D), _row_spec(1),
                  _full_spec((4, D, D)), _full_spec((4, D)),
                  _full_spec((4, D, D)), _full_spec((4, D)),
                  _full_spec((2, D)), _full_spec((1, 1)),
                  _full_spec((2, D)), _full_spec((1, 1)),
                  _full_spec((D, D)), _full_spec((1, D)),
                  _full_spec((D, D)), _full_spec((1, D)),
                  _full_spec((D, D)), _full_spec((1, D)),
                  _full_spec((D, D)), _full_spec((1, D))],
        out_specs=[_row_spec(D)],
        out_shape=[jax.ShapeDtypeStruct((NPAD, D), _F32)],
    )(w0, w1, u2, ax, h1, dinv, Wh, bh, Wx, bx, gf, gfb, gu, gub,
      f1w, f1b, f2w, f2b, c1w, c1b, c2w, c2b)


def kernel(x, edge_index, conv_h_W, conv_h_b, conv_x_W, conv_x_b,
           gate_f_W, gate_f_b, gate_u_W, gate_u_b,
           ff1_W, ff1_b, ff2_W, ff2_b, clf1_W, clf1_b, clf2_W, clf2_b):
    pad = jnp.full((EPAD - E,), N, dtype=jnp.int32)
    src3 = jnp.concatenate([edge_index[0], pad]).reshape(NW, CPT, CHUNK)
    dst3 = jnp.concatenate([edge_index[1], pad]).reshape(NW, CPT, CHUNK)
    zeros = jnp.zeros((NPAD, D), _F32)
    ones_full = jnp.ones((NPAD, D), _F32)
    x_pad = jnp.concatenate([x, jnp.zeros((NPAD - N, D), _F32)])
    gf = gate_f_W[:, 0].reshape(2, D)
    gu = gate_u_W[:, 0].reshape(2, D)
    gfb = gate_f_b.reshape(1, 1)
    gub = gate_u_b.reshape(1, 1)
    f1b = ff1_b.reshape(2, 1, D)
    f2b = ff2_b.reshape(2, 1, D)
    c1b = clf1_b.reshape(1, D)
    c2b = clf2_b.reshape(1, D)

    sc_pass = _sc_kernels()

    degp = sc_pass(ones_full, src3, dst3, zeros)

    u1, dinv = _prep_call(degp[0], degp[1], x_pad)

    w1p = sc_pass(u1, src3, dst3, zeros)

    ax, h1, u2 = _layer0_call(
        w1p[0], w1p[1], u1, dinv,
        conv_x_W[0], conv_x_b[0], conv_h_b[0],
        gf, gfb, gu, gub, ff1_W[0], f1b[0], ff2_W[0], f2b[0])

    w2p = sc_pass(u2, src3, dst3, zeros)

    (outp,) = _layer1_call(
        w2p[0], w2p[1], u2, ax, h1, dinv,
        conv_h_W[1], conv_h_b[1], conv_x_W[1], conv_x_b[1],
        gf, gfb, gu, gub, ff1_W[1], f1b[1], ff2_W[1], f2b[1],
        clf1_W, c1b, clf2_W, c2b)
    return outp[:N]
